# baseline (speedup 1.0000x reference)
import jax
import jax.numpy as jnp
from jax import lax
from jax.experimental import pallas as pl
from jax.experimental.pallas import tpu as pltpu
from jax.experimental.pallas import tpu_sc as plsc

NY, NX, C = 496, 432, 64
NB = 4
NPTS = 48000
NYNX = NY * NX
CHUNK = 2304
NCH_B = NYNX // CHUNK
NPAD = 49152
PT = NPAD // 16
NG = PT // 16
WCELLS = CHUNK // 16
HCELLS = WCELLS // 3
RW = 128


def _body(feat, coorsT, out, coor_v, cells_v, idx_v,
          rows_v, wb_v, tr_v, zero_v, canvas_s, sem):
    sc = lax.axis_index("c")
    tid = lax.axis_index("s")
    iota = lax.iota(jnp.int32, 16)
    zf = jnp.zeros((16,), jnp.float32)
    base = tid * PT

    pltpu.sync_copy(coorsT.at[:, pl.ds(base, PT)], coor_v)
    sc_base = sc * (2 * NYNX)

    def cellg(i, _):
        off = i * 16
        b = coor_v[0, pl.ds(off, 16)]
        y = coor_v[2, pl.ds(off, 16)]
        x = coor_v[3, pl.ds(off, 16)]
        cells_v[pl.ds(off, 16)] = b * NYNX + y * NX + x - sc_base
        return 0

    lax.fori_loop(0, NG, cellg, 0)

    def zrow(i, _):
        zero_v[i // 8, pl.ds((i % 8) * 16, 16)] = zf
        return 0

    lax.fori_loop(0, HCELLS * 8, zrow, 0)

    def chunk(ci, _):
        lo = ci * CHUNK
        bb = sc * 2 + ci // NCH_B
        cb = (ci % NCH_B) * CHUNK

        for h in range(3):
            pltpu.sync_copy(
                zero_v,
                canvas_s.at[pl.ds(tid * WCELLS + h * HCELLS, HCELLS)])
        plsc.subcore_barrier()

        def scang(i, cur):
            cl = cells_v[pl.ds(i * 16, 16)]
            m = jnp.logical_and(cl >= lo, cl < lo + CHUNK)
            mi = m.astype(jnp.int32)
            pos = cur + plsc.cumsum(mi) - 1
            plsc.store_scatter(idx_v, [pos], i * 16 + iota, mask=m)
            return cur + plsc.all_reduce_population_count(m)[0]

        ncomp = lax.fori_loop(0, NG, scang, jnp.int32(0))
        nsub = (ncomp + 15) // 16

        def sub(j, _):
            off = j * 16
            valid = (off + iota) < ncomp
            ids = jnp.where(valid, idx_v[pl.ds(off, 16)], 0)
            cl = plsc.load_gather(cells_v, [ids])
            gid = base + ids
            pltpu.async_copy(feat.at[gid], rows_v, sem).wait()
            dst = jnp.where(valid, cl - lo, jnp.int32(CHUNK))
            pltpu.sync_copy(rows_v, canvas_s.at[dst], add=True)
            return 0

        lax.fori_loop(0, nsub, sub, 0)
        plsc.subcore_barrier()

        for h in range(3):
            wlo = tid * WCELLS + h * HCELLS
            pltpu.sync_copy(canvas_s.at[pl.ds(wlo, HCELLS)], wb_v)

            def cellk(k, _):
                kcol = jnp.full((16,), k, jnp.int32)
                for g in range(5):
                    v = wb_v[k, pl.ds(g * 16, 16)]
                    plsc.store_scatter(tr_v, [g * 16 + iota, kcol], v)
                return 0

            lax.fori_loop(0, HCELLS, cellk, 0)

            def scaler(r, _):
                cnt = tr_v[C, pl.ds(r * 16, 16)]
                rcp = 1.0 / jnp.maximum(cnt, 1.0)

                def sch(ch, _):
                    sl = tr_v[ch, pl.ds(r * 16, 16)]
                    tr_v[ch, pl.ds(r * 16, 16)] = sl * rcp
                    return 0

                lax.fori_loop(0, C, sch, 0)
                return 0

            lax.fori_loop(0, HCELLS // 16, scaler, 0)
            pltpu.sync_copy(
                tr_v.at[pl.ds(0, C)],
                out.at[pl.ds(bb * C, C), pl.ds(cb + wlo, HCELLS)])
        return 0

    lax.fori_loop(0, 2 * NCH_B, chunk, 0)


def _make_kernel():
    mesh = plsc.VectorSubcoreMesh(core_axis_name="c", subcore_axis_name="s")
    return pl.kernel(
        _body,
        out_type=jax.ShapeDtypeStruct((NB * C, NYNX), jnp.float32),
        mesh=mesh,
        compiler_params=pltpu.CompilerParams(
            use_tc_tiling_on_sc=False, needs_layout_passes=False),
        scratch_types=[
            pltpu.VMEM((4, PT), jnp.int32),
            pltpu.VMEM((PT, ), jnp.int32),
            pltpu.VMEM((PT + 16, ), jnp.int32),
            pltpu.VMEM((16, RW), jnp.float32),
            pltpu.VMEM((HCELLS, RW), jnp.float32),
            pltpu.VMEM((C + 16, HCELLS), jnp.float32),
            pltpu.VMEM((HCELLS, RW), jnp.float32),
            pltpu.VMEM_SHARED((CHUNK + 16, RW), jnp.float32),
            pltpu.SemaphoreType.DMA,
        ],
    )


@jax.jit
def _run(feat, coorsT):
    return _make_kernel()(feat, coorsT)


def kernel(voxel_features, coors, batch_size):
    n = voxel_features.shape[0]
    feat = jnp.concatenate(
        [voxel_features.astype(jnp.float32),
         jnp.ones((n, 1), jnp.float32),
         jnp.zeros((n, RW - C - 1), jnp.float32)], axis=1)
    feat = jnp.pad(feat, ((0, NPAD - n), (0, 0)))
    coorsT = jnp.transpose(coors.astype(jnp.int32))
    coorsT = jnp.pad(coorsT, ((0, 0), (0, NPAD - n)), constant_values=NB)
    out = _run(feat, coorsT)
    return out.reshape(NB, C, NY, NX)


# speedup vs baseline: 1.0319x; 1.0319x over previous
import jax
import jax.numpy as jnp
from jax import lax
from jax.experimental import pallas as pl
from jax.experimental.pallas import tpu as pltpu
from jax.experimental.pallas import tpu_sc as plsc

NY, NX, C = 496, 432, 64
NB = 4
NPTS = 48000
NYNX = NY * NX
CHUNK = 2304
NCH_B = NYNX // CHUNK
NPAD = 49152
PT = NPAD // 16
NG = PT // 16
WCELLS = CHUNK // 16
HCELLS = WCELLS // 3
RW = 80


def _body(feat, coorsT, out, coor_v, cells_v, idx_v,
          rows_v, wb_v, tr_v, zero_v, canvas_s, sem):
    sc = lax.axis_index("c")
    tid = lax.axis_index("s")
    iota = lax.iota(jnp.int32, 16)
    zf = jnp.zeros((16,), jnp.float32)
    base = tid * PT

    pltpu.sync_copy(coorsT.at[:, pl.ds(base, PT)], coor_v)
    sc_base = sc * (2 * NYNX)

    def cellg(i, _):
        off = i * 16
        b = coor_v[0, pl.ds(off, 16)]
        y = coor_v[2, pl.ds(off, 16)]
        x = coor_v[3, pl.ds(off, 16)]
        cells_v[pl.ds(off, 16)] = b * NYNX + y * NX + x - sc_base
        return 0

    lax.fori_loop(0, NG, cellg, 0)

    nrw = RW // 16

    def zrow(i, _):
        zero_v[i // nrw, pl.ds((i % nrw) * 16, 16)] = zf
        return 0

    lax.fori_loop(0, HCELLS * nrw, zrow, 0)

    def chunk(ci, _):
        lo = ci * CHUNK
        bb = sc * 2 + ci // NCH_B
        cb = (ci % NCH_B) * CHUNK

        for h in range(3):
            pltpu.sync_copy(
                zero_v,
                canvas_s.at[pl.ds(tid * WCELLS + h * HCELLS, HCELLS)])
        plsc.subcore_barrier()

        def scang(i, cur):
            cl = cells_v[pl.ds(i * 16, 16)]
            m = jnp.logical_and(cl >= lo, cl < lo + CHUNK)
            mi = m.astype(jnp.int32)
            pos = cur + plsc.cumsum(mi) - 1
            plsc.store_scatter(idx_v, [pos], i * 16 + iota, mask=m)
            return cur + plsc.all_reduce_population_count(m)[0]

        ncomp = lax.fori_loop(0, NG, scang, jnp.int32(0))
        nsub = (ncomp + 15) // 16

        def sub(j, _):
            off = j * 16
            valid = (off + iota) < ncomp
            ids = jnp.where(valid, idx_v[pl.ds(off, 16)], 0)
            cl = plsc.load_gather(cells_v, [ids])
            gid = base + ids
            pltpu.async_copy(feat.at[gid], rows_v, sem).wait()
            dst = jnp.where(valid, cl - lo, jnp.int32(CHUNK))
            pltpu.sync_copy(rows_v, canvas_s.at[dst], add=True)
            return 0

        lax.fori_loop(0, nsub, sub, 0)
        plsc.subcore_barrier()

        for h in range(3):
            wlo = tid * WCELLS + h * HCELLS
            pltpu.sync_copy(canvas_s.at[pl.ds(wlo, HCELLS)], wb_v)

            def cellk(k, _):
                kcol = jnp.full((16,), k, jnp.int32)
                for g in range(5):
                    v = wb_v[k, pl.ds(g * 16, 16)]
                    plsc.store_scatter(tr_v, [g * 16 + iota, kcol], v)
                return 0

            lax.fori_loop(0, HCELLS, cellk, 0)

            def scaler(r, _):
                cnt = tr_v[C, pl.ds(r * 16, 16)]
                rcp = 1.0 / jnp.maximum(cnt, 1.0)

                def sch(ch, _):
                    sl = tr_v[ch, pl.ds(r * 16, 16)]
                    tr_v[ch, pl.ds(r * 16, 16)] = sl * rcp
                    return 0

                lax.fori_loop(0, C, sch, 0)
                return 0

            lax.fori_loop(0, HCELLS // 16, scaler, 0)
            pltpu.sync_copy(
                tr_v.at[pl.ds(0, C)],
                out.at[pl.ds(bb * C, C), pl.ds(cb + wlo, HCELLS)])
        return 0

    lax.fori_loop(0, 2 * NCH_B, chunk, 0)


def _make_kernel():
    mesh = plsc.VectorSubcoreMesh(core_axis_name="c", subcore_axis_name="s")
    return pl.kernel(
        _body,
        out_type=jax.ShapeDtypeStruct((NB * C, NYNX), jnp.float32),
        mesh=mesh,
        compiler_params=pltpu.CompilerParams(
            use_tc_tiling_on_sc=False, needs_layout_passes=False),
        scratch_types=[
            pltpu.VMEM((4, PT), jnp.int32),
            pltpu.VMEM((PT, ), jnp.int32),
            pltpu.VMEM((PT + 16, ), jnp.int32),
            pltpu.VMEM((16, RW), jnp.float32),
            pltpu.VMEM((HCELLS, RW), jnp.float32),
            pltpu.VMEM((C + 16, HCELLS), jnp.float32),
            pltpu.VMEM((HCELLS, RW), jnp.float32),
            pltpu.VMEM_SHARED((CHUNK + 16, RW), jnp.float32),
            pltpu.SemaphoreType.DMA,
        ],
    )


@jax.jit
def _run(feat, coorsT):
    return _make_kernel()(feat, coorsT)


def kernel(voxel_features, coors, batch_size):
    n = voxel_features.shape[0]
    feat = jnp.concatenate(
        [voxel_features.astype(jnp.float32),
         jnp.ones((n, 1), jnp.float32),
         jnp.zeros((n, RW - C - 1), jnp.float32)], axis=1)
    feat = jnp.pad(feat, ((0, NPAD - n), (0, 0)))
    coorsT = jnp.transpose(coors.astype(jnp.int32))
    coorsT = jnp.pad(coorsT, ((0, 0), (0, NPAD - n)), constant_values=NB)
    out = _run(feat, coorsT)
    return out.reshape(NB, C, NY, NX)


# speedup vs baseline: 1.2033x; 1.1661x over previous
import jax
import jax.numpy as jnp
from jax import lax
from jax.experimental import pallas as pl
from jax.experimental.pallas import tpu as pltpu
from jax.experimental.pallas import tpu_sc as plsc

NY, NX, C = 496, 432, 64
NB = 4
NPTS = 48000
NYNX = NY * NX
CHUNK = 6912
NCH_B = NYNX // CHUNK
NPAD = 49152
PT = NPAD // 16
NG = PT // 16
WCELLS = CHUNK // 16
HCELLS = WCELLS // 3
RW = 80


def _body(feat, coorsT, out, coor_v, cells_v, idx_v,
          rows_v, wb_v, tr_v, zero_v, canvas_s, sem):
    sc = lax.axis_index("c")
    tid = lax.axis_index("s")
    iota = lax.iota(jnp.int32, 16)
    zf = jnp.zeros((16,), jnp.float32)
    base = tid * PT

    pltpu.sync_copy(coorsT.at[:, pl.ds(base, PT)], coor_v)
    sc_base = sc * (2 * NYNX)

    def cellg(i, _):
        off = i * 16
        b = coor_v[0, pl.ds(off, 16)]
        y = coor_v[2, pl.ds(off, 16)]
        x = coor_v[3, pl.ds(off, 16)]
        cells_v[pl.ds(off, 16)] = b * NYNX + y * NX + x - sc_base
        return 0

    lax.fori_loop(0, NG, cellg, 0)

    nrw = RW // 16

    def zrow(i, _):
        zero_v[i // nrw, pl.ds((i % nrw) * 16, 16)] = zf
        return 0

    lax.fori_loop(0, HCELLS * nrw, zrow, 0)

    def chunk(ci, _):
        lo = ci * CHUNK
        bb = sc * 2 + ci // NCH_B
        cb = (ci % NCH_B) * CHUNK

        for h in range(3):
            pltpu.sync_copy(
                zero_v,
                canvas_s.at[pl.ds(tid * WCELLS + h * HCELLS, HCELLS)])
        plsc.subcore_barrier()

        def scang(i, cur):
            cl = cells_v[pl.ds(i * 16, 16)]
            m = jnp.logical_and(cl >= lo, cl < lo + CHUNK)
            mi = m.astype(jnp.int32)
            pos = cur + plsc.cumsum(mi) - 1
            plsc.store_scatter(idx_v, [pos], i * 16 + iota, mask=m)
            return cur + plsc.all_reduce_population_count(m)[0]

        ncomp = lax.fori_loop(0, NG, scang, jnp.int32(0))
        nsub = (ncomp + 15) // 16

        def sub(j, _):
            off = j * 16
            valid = (off + iota) < ncomp
            ids = jnp.where(valid, idx_v[pl.ds(off, 16)], 0)
            cl = plsc.load_gather(cells_v, [ids])
            gid = base + ids
            pltpu.async_copy(feat.at[gid], rows_v, sem).wait()
            dst = jnp.where(valid, cl - lo, jnp.int32(CHUNK))
            pltpu.sync_copy(rows_v, canvas_s.at[dst], add=True)
            return 0

        lax.fori_loop(0, nsub, sub, 0)
        plsc.subcore_barrier()

        for h in range(3):
            wlo = tid * WCELLS + h * HCELLS
            pltpu.sync_copy(canvas_s.at[pl.ds(wlo, HCELLS)], wb_v)

            def cellk(k, _):
                kcol = jnp.full((16,), k, jnp.int32)
                for g in range(5):
                    v = wb_v[k, pl.ds(g * 16, 16)]
                    plsc.store_scatter(tr_v, [g * 16 + iota, kcol], v)
                return 0

            lax.fori_loop(0, HCELLS, cellk, 0)

            def scaler(r, _):
                cnt = tr_v[C, pl.ds(r * 16, 16)]
                rcp = 1.0 / jnp.maximum(cnt, 1.0)

                def sch(ch, _):
                    sl = tr_v[ch, pl.ds(r * 16, 16)]
                    tr_v[ch, pl.ds(r * 16, 16)] = sl * rcp
                    return 0

                lax.fori_loop(0, C, sch, 0)
                return 0

            lax.fori_loop(0, HCELLS // 16, scaler, 0)
            pltpu.sync_copy(
                tr_v.at[pl.ds(0, C)],
                out.at[pl.ds(bb * C, C), pl.ds(cb + wlo, HCELLS)])
        return 0

    lax.fori_loop(0, 2 * NCH_B, chunk, 0)


def _make_kernel():
    mesh = plsc.VectorSubcoreMesh(core_axis_name="c", subcore_axis_name="s")
    return pl.kernel(
        _body,
        out_type=jax.ShapeDtypeStruct((NB * C, NYNX), jnp.float32),
        mesh=mesh,
        compiler_params=pltpu.CompilerParams(
            use_tc_tiling_on_sc=False, needs_layout_passes=False),
        scratch_types=[
            pltpu.VMEM((4, PT), jnp.int32),
            pltpu.VMEM((PT, ), jnp.int32),
            pltpu.VMEM((PT + 16, ), jnp.int32),
            pltpu.VMEM((16, RW), jnp.float32),
            pltpu.VMEM((HCELLS, RW), jnp.float32),
            pltpu.VMEM((C + 16, HCELLS), jnp.float32),
            pltpu.VMEM((HCELLS, RW), jnp.float32),
            pltpu.VMEM_SHARED((CHUNK + 16, RW), jnp.float32),
            pltpu.SemaphoreType.DMA,
        ],
    )


@jax.jit
def _run(feat, coorsT):
    return _make_kernel()(feat, coorsT)


def kernel(voxel_features, coors, batch_size):
    n = voxel_features.shape[0]
    feat = jnp.concatenate(
        [voxel_features.astype(jnp.float32),
         jnp.ones((n, 1), jnp.float32),
         jnp.zeros((n, RW - C - 1), jnp.float32)], axis=1)
    feat = jnp.pad(feat, ((0, NPAD - n), (0, 0)))
    coorsT = jnp.transpose(coors.astype(jnp.int32))
    coorsT = jnp.pad(coorsT, ((0, 0), (0, NPAD - n)), constant_values=NB)
    out = _run(feat, coorsT)
    return out.reshape(NB, C, NY, NX)
